# Initial kernel scaffold; baseline (speedup 1.0000x reference)
import jax
import jax.numpy as jnp
from jax.experimental import pallas as pl


def kernel(t, enc):
    raise NotImplementedError("write your pallas kernel here")


# speedup vs baseline: 2.9093x; 2.9093x over previous
import functools

import jax
import jax.numpy as jnp
from jax import lax
from jax.experimental import pallas as pl
from jax.experimental.pallas import tpu as pltpu
from jax.experimental.pallas import tpu_sc as plsc

EMBED_DIM = 128
NUM_T = 4096 * 50
NC, NS = 2, 16
NW = NC * NS
B_PER_W = NUM_T // NW
CHUNK = 128
NCHUNK = B_PER_W // CHUNK


def _gather_body(idx_hbm, enc_hbm, out_hbm, idx_v, rows_v, sem):
    wid = lax.axis_index("s") * NC + lax.axis_index("c")
    base = wid * B_PER_W
    pltpu.sync_copy(idx_hbm.at[wid], idx_v)

    def step(j, carry):
        pltpu.async_copy(enc_hbm.at[idx_v.at[j]], rows_v, sem).wait()
        pltpu.sync_copy(rows_v, out_hbm.at[pl.ds(base + j * CHUNK, CHUNK)])
        return carry

    lax.fori_loop(0, NCHUNK, step, 0)


_gather = pl.kernel(
    _gather_body,
    out_type=jax.ShapeDtypeStruct((NUM_T, EMBED_DIM), jnp.float32),
    mesh=plsc.VectorSubcoreMesh(core_axis_name="c", subcore_axis_name="s"),
    scratch_types=[
        pltpu.VMEM((NCHUNK, CHUNK), jnp.int32),
        pltpu.VMEM((CHUNK, EMBED_DIM), jnp.float32),
        pltpu.SemaphoreType.DMA,
    ],
)


@jax.jit
def kernel(t, enc):
    b, s = t.shape
    idx = t.astype(jnp.int32).reshape(NW, NCHUNK, CHUNK)
    out = _gather(idx, enc)
    return out.reshape(b, s, EMBED_DIM)


# speedup vs baseline: 3.2689x; 1.1236x over previous
import functools

import jax
import jax.numpy as jnp
from jax import lax
from jax.experimental import pallas as pl
from jax.experimental.pallas import tpu as pltpu
from jax.experimental.pallas import tpu_sc as plsc

EMBED_DIM = 128
NUM_T = 4096 * 50
NC, NS = 2, 16
NW = NC * NS
B_PER_W = NUM_T // NW
CHUNK = 128
NCHUNK = B_PER_W // CHUNK


NBUF = 5
NITER = NCHUNK // NBUF


def _gather_body(idx_hbm, enc_hbm, out_hbm, idx_v, bufs, sems):
    wid = lax.axis_index("s") * NC + lax.axis_index("c")
    base = wid * B_PER_W
    pltpu.sync_copy(idx_hbm.at[wid], idx_v)

    for b in range(NBUF):
        pltpu.async_copy(enc_hbm.at[idx_v.at[b]], bufs[b], sems[b])

    def step(i, carry):
        for b in range(NBUF):
            j = i * NBUF + b
            pltpu.make_async_copy(enc_hbm.at[idx_v.at[j]], bufs[b], sems[b]).wait()
            pltpu.sync_copy(bufs[b], out_hbm.at[pl.ds(base + j * CHUNK, CHUNK)])

            @pl.when(i < NITER - 1)
            def _():
                pltpu.async_copy(enc_hbm.at[idx_v.at[j + NBUF]], bufs[b], sems[b])

        return carry

    lax.fori_loop(0, NITER, step, 0)


_gather = pl.kernel(
    _gather_body,
    out_type=jax.ShapeDtypeStruct((NUM_T, EMBED_DIM), jnp.float32),
    mesh=plsc.VectorSubcoreMesh(core_axis_name="c", subcore_axis_name="s"),
    scratch_types=[
        pltpu.VMEM((NCHUNK, CHUNK), jnp.int32),
        [pltpu.VMEM((CHUNK, EMBED_DIM), jnp.float32) for _ in range(NBUF)],
        [pltpu.SemaphoreType.DMA for _ in range(NBUF)],
    ],
)


@jax.jit
def kernel(t, enc):
    b, s = t.shape
    idx = t.astype(jnp.int32).reshape(NW, NCHUNK, CHUNK)
    out = _gather(idx, enc)
    return out.reshape(b, s, EMBED_DIM)


# speedup vs baseline: 3.6595x; 1.1195x over previous
import functools

import jax
import jax.numpy as jnp
from jax import lax
from jax.experimental import pallas as pl
from jax.experimental.pallas import tpu as pltpu
from jax.experimental.pallas import tpu_sc as plsc

EMBED_DIM = 128
NUM_T = 4096 * 50
NC, NS = 2, 16
NW = NC * NS
B_PER_W = NUM_T // NW
CHUNK = 128
NCHUNK = B_PER_W // CHUNK


NBUF = 2
NITER = NCHUNK // NBUF


TABLE_ROWS = 8192
STAGE_ROWS = TABLE_ROWS // NS


def _gather_body(idx_hbm, enc_hbm, out_hbm, idx_v, enc_sp, bufs, sems):
    cid = lax.axis_index("c")
    sid = lax.axis_index("s")
    wid = sid * NC + cid
    base = wid * B_PER_W

    pltpu.sync_copy(
        enc_hbm.at[pl.ds(sid * STAGE_ROWS, STAGE_ROWS)],
        enc_sp.at[pl.ds(sid * STAGE_ROWS, STAGE_ROWS)],
    )
    pltpu.sync_copy(idx_hbm.at[wid], idx_v)
    plsc.subcore_barrier()

    for b in range(NBUF):
        pltpu.async_copy(enc_sp.at[idx_v.at[b]], bufs[b], sems[b])

    def step(i, carry):
        for b in range(NBUF):
            j = i * NBUF + b
            pltpu.make_async_copy(enc_sp.at[idx_v.at[j]], bufs[b], sems[b]).wait()
            pltpu.sync_copy(bufs[b], out_hbm.at[pl.ds(base + j * CHUNK, CHUNK)])

            @pl.when(i < NITER - 1)
            def _():
                pltpu.async_copy(enc_sp.at[idx_v.at[j + NBUF]], bufs[b], sems[b])

        return carry

    lax.fori_loop(0, NITER, step, 0)


_gather = pl.kernel(
    _gather_body,
    out_type=jax.ShapeDtypeStruct((NUM_T, EMBED_DIM), jnp.float32),
    mesh=plsc.VectorSubcoreMesh(core_axis_name="c", subcore_axis_name="s"),
    scratch_types=[
        pltpu.VMEM((NCHUNK, CHUNK), jnp.int32),
        pltpu.VMEM_SHARED((TABLE_ROWS, EMBED_DIM), jnp.float32),
        [pltpu.VMEM((CHUNK, EMBED_DIM), jnp.float32) for _ in range(NBUF)],
        [pltpu.SemaphoreType.DMA for _ in range(NBUF)],
    ],
)


@jax.jit
def kernel(t, enc):
    b, s = t.shape
    idx = t.astype(jnp.int32).reshape(NW, NCHUNK, CHUNK)
    out = _gather(idx, enc)
    return out.reshape(b, s, EMBED_DIM)


# speedup vs baseline: 14.4185x; 3.9400x over previous
import functools

import jax
import jax.numpy as jnp
from jax import lax
from jax.experimental import pallas as pl
from jax.experimental.pallas import tpu as pltpu
from jax.experimental.pallas import tpu_sc as plsc

EMBED_DIM = 128
NUM_T = 4096 * 50
NC, NS = 2, 16
NW = NC * NS
B_PER_W = NUM_T // NW
CHUNK = 128
NCHUNK = B_PER_W // CHUNK


NBUF = 2
NITER = NCHUNK // NBUF


TABLE_ROWS = 8192
STAGE_ROWS = TABLE_ROWS // NS


def _gather_body(idx_hbm, enc_hbm, out_hbm, idx_v, enc_sp, bufs, sems):
    cid = lax.axis_index("c")
    sid = lax.axis_index("s")
    wid = sid * NC + cid
    base = wid * B_PER_W

    pltpu.sync_copy(
        enc_hbm.at[pl.ds(sid * STAGE_ROWS, STAGE_ROWS)],
        enc_sp.at[pl.ds(sid * STAGE_ROWS, STAGE_ROWS)],
    )
    pltpu.sync_copy(idx_hbm.at[wid], idx_v)
    plsc.subcore_barrier()

    for b in range(NBUF):
        pltpu.async_copy(enc_sp.at[idx_v.at[b]], bufs[b], sems[b])

    def step(i, carry):
        for b in range(NBUF):
            j = i * NBUF + b
            pltpu.make_async_copy(enc_sp.at[idx_v.at[j]], bufs[b], sems[b]).wait()
            pltpu.sync_copy(bufs[b], out_hbm.at[pl.ds(base + j * CHUNK, CHUNK)])

            @pl.when(i < NITER - 1)
            def _():
                pltpu.async_copy(enc_sp.at[idx_v.at[j + NBUF]], bufs[b], sems[b])

        return carry

    lax.fori_loop(0, NITER, step, 0)


_gather = pl.kernel(
    _gather_body,
    out_type=jax.ShapeDtypeStruct((NUM_T, EMBED_DIM), jnp.float32),
    mesh=plsc.VectorSubcoreMesh(core_axis_name="c", subcore_axis_name="s"),
    scratch_types=[
        pltpu.VMEM((NCHUNK, CHUNK), jnp.int32),
        pltpu.VMEM_SHARED((TABLE_ROWS, EMBED_DIM), jnp.float32),
        [pltpu.VMEM((CHUNK, EMBED_DIM), jnp.float32) for _ in range(NBUF)],
        [pltpu.SemaphoreType.DMA for _ in range(NBUF)],
    ],
)


@jax.jit
def kernel(t, enc):
    b, s = t.shape
    idx = t.T.astype(jnp.int32).reshape(NW, NCHUNK, CHUNK)
    out = _gather(idx, enc)
    return out.reshape(s, b, EMBED_DIM).transpose(1, 0, 2)


# speedup vs baseline: 14.5589x; 1.0097x over previous
import functools

import jax
import jax.numpy as jnp
from jax import lax
from jax.experimental import pallas as pl
from jax.experimental.pallas import tpu as pltpu
from jax.experimental.pallas import tpu_sc as plsc

EMBED_DIM = 128
NUM_T = 4096 * 50
NC, NS = 2, 16
NW = NC * NS
B_PER_W = NUM_T // NW
CHUNK = 80
NCHUNK = B_PER_W // CHUNK


NBUF = 4
GAHEAD = 2
NITER = NCHUNK // NBUF


TABLE_ROWS = 8192
STAGE_ROWS = TABLE_ROWS // NS


def _gather_body(idx_hbm, enc_hbm, out_hbm, idx_v, enc_sp, bufs, gsems, ssems):
    cid = lax.axis_index("c")
    sid = lax.axis_index("s")
    wid = sid * NC + cid
    base = wid * B_PER_W

    pltpu.sync_copy(
        enc_hbm.at[pl.ds(sid * STAGE_ROWS, STAGE_ROWS)],
        enc_sp.at[pl.ds(sid * STAGE_ROWS, STAGE_ROWS)],
    )
    pltpu.sync_copy(idx_hbm.at[wid], idx_v)
    plsc.subcore_barrier()

    for b in range(GAHEAD):
        pltpu.async_copy(enc_sp.at[idx_v.at[b]], bufs[b], gsems[b])

    def step(i, carry):
        for b in range(NBUF):
            j = i * NBUF + b
            k = j + GAHEAD
            bk = (b + GAHEAD) % NBUF
            pltpu.make_async_copy(enc_sp.at[idx_v.at[j]], bufs[b], gsems[b]).wait()
            pltpu.async_copy(
                bufs[b], out_hbm.at[pl.ds(base + j * CHUNK, CHUNK)], ssems[b]
            )

            @pl.when(k < NCHUNK)
            def _():
                @pl.when(k >= NBUF)
                def _():
                    prev = k - NBUF
                    pltpu.make_async_copy(
                        bufs[bk],
                        out_hbm.at[pl.ds(base + prev * CHUNK, CHUNK)],
                        ssems[bk],
                    ).wait()

                pltpu.async_copy(enc_sp.at[idx_v.at[k]], bufs[bk], gsems[bk])

        return carry

    lax.fori_loop(0, NITER, step, 0)

    for b in range(NBUF):
        last = NCHUNK - NBUF + b
        pltpu.make_async_copy(
            bufs[b], out_hbm.at[pl.ds(base + last * CHUNK, CHUNK)], ssems[b]
        ).wait()


_gather = pl.kernel(
    _gather_body,
    out_type=jax.ShapeDtypeStruct((NUM_T, EMBED_DIM), jnp.float32),
    mesh=plsc.VectorSubcoreMesh(core_axis_name="c", subcore_axis_name="s"),
    scratch_types=[
        pltpu.VMEM((NCHUNK, CHUNK), jnp.int32),
        pltpu.VMEM_SHARED((TABLE_ROWS, EMBED_DIM), jnp.float32),
        [pltpu.VMEM((CHUNK, EMBED_DIM), jnp.float32) for _ in range(NBUF)],
        [pltpu.SemaphoreType.DMA for _ in range(NBUF)],
        [pltpu.SemaphoreType.DMA for _ in range(NBUF)],
    ],
)


@jax.jit
def kernel(t, enc):
    b, s = t.shape
    idx = t.T.astype(jnp.int32).reshape(NW, NCHUNK, CHUNK)
    out = _gather(idx, enc)
    return out.reshape(s, b, EMBED_DIM).transpose(1, 0, 2)


# speedup vs baseline: 14.7500x; 1.0131x over previous
import functools

import jax
import jax.numpy as jnp
from jax import lax
from jax.experimental import pallas as pl
from jax.experimental.pallas import tpu as pltpu
from jax.experimental.pallas import tpu_sc as plsc

EMBED_DIM = 128
NUM_T = 4096 * 50
NC, NS = 2, 16
NW = NC * NS
B_PER_W = NUM_T // NW
CHUNK = 80
NCHUNK = B_PER_W // CHUNK


NBUF = 4
GAHEAD = 2
NITER = NCHUNK // NBUF


TABLE_ROWS = 8192
STAGE_ROWS = TABLE_ROWS // NS


SWITCH = NBUF


def _gather_body(idx_hbm, enc_hbm, out_hbm, idx_v, enc_sp, bufs, gsems, ssems,
                 stage_sem):
    cid = lax.axis_index("c")
    sid = lax.axis_index("s")
    wid = sid * NC + cid
    base = wid * B_PER_W

    stage_src = enc_hbm.at[pl.ds(sid * STAGE_ROWS, STAGE_ROWS)]
    stage_dst = enc_sp.at[pl.ds(sid * STAGE_ROWS, STAGE_ROWS)]
    pltpu.async_copy(stage_src, stage_dst, stage_sem)
    pltpu.sync_copy(idx_hbm.at[wid], idx_v)

    for b in range(GAHEAD):
        pltpu.async_copy(enc_hbm.at[idx_v.at[b]], bufs[b], gsems[b])

    def step(i, carry):
        for b in range(NBUF):
            j = i * NBUF + b
            k = j + GAHEAD
            bk = (b + GAHEAD) % NBUF

            @pl.when(j < SWITCH)
            def _():
                pltpu.make_async_copy(
                    enc_hbm.at[idx_v.at[j]], bufs[b], gsems[b]
                ).wait()

            @pl.when(j >= SWITCH)
            def _():
                pltpu.make_async_copy(
                    enc_sp.at[idx_v.at[j]], bufs[b], gsems[b]
                ).wait()

            pltpu.async_copy(
                bufs[b], out_hbm.at[pl.ds(base + j * CHUNK, CHUNK)], ssems[b]
            )

            @pl.when(j == SWITCH - GAHEAD)
            def _():
                pltpu.make_async_copy(stage_src, stage_dst, stage_sem).wait()
                plsc.subcore_barrier()

            @pl.when(k < NCHUNK)
            def _():
                @pl.when(k >= NBUF)
                def _():
                    prev = k - NBUF
                    pltpu.make_async_copy(
                        bufs[bk],
                        out_hbm.at[pl.ds(base + prev * CHUNK, CHUNK)],
                        ssems[bk],
                    ).wait()

                @pl.when(k < SWITCH)
                def _():
                    pltpu.async_copy(enc_hbm.at[idx_v.at[k]], bufs[bk], gsems[bk])

                @pl.when(k >= SWITCH)
                def _():
                    pltpu.async_copy(enc_sp.at[idx_v.at[k]], bufs[bk], gsems[bk])

        return carry

    lax.fori_loop(0, NITER, step, 0)

    for b in range(NBUF):
        last = NCHUNK - NBUF + b
        pltpu.make_async_copy(
            bufs[b], out_hbm.at[pl.ds(base + last * CHUNK, CHUNK)], ssems[b]
        ).wait()


_gather = pl.kernel(
    _gather_body,
    out_type=jax.ShapeDtypeStruct((NUM_T, EMBED_DIM), jnp.float32),
    mesh=plsc.VectorSubcoreMesh(core_axis_name="c", subcore_axis_name="s"),
    scratch_types=[
        pltpu.VMEM((NCHUNK, CHUNK), jnp.int32),
        pltpu.VMEM_SHARED((TABLE_ROWS, EMBED_DIM), jnp.float32),
        [pltpu.VMEM((CHUNK, EMBED_DIM), jnp.float32) for _ in range(NBUF)],
        [pltpu.SemaphoreType.DMA for _ in range(NBUF)],
        [pltpu.SemaphoreType.DMA for _ in range(NBUF)],
        pltpu.SemaphoreType.DMA,
    ],
)


@jax.jit
def kernel(t, enc):
    b, s = t.shape
    idx = t.T.astype(jnp.int32).reshape(NW, NCHUNK, CHUNK)
    out = _gather(idx, enc)
    return out.reshape(s, b, EMBED_DIM).transpose(1, 0, 2)
